# Initial kernel scaffold; baseline (speedup 1.0000x reference)
import jax
import jax.numpy as jnp
from jax.experimental import pallas as pl


def kernel(x, edge_index, batch, W1, b1, W2, b2, Wil, bil, Whl1, bhl1, Wol, bol):
    raise NotImplementedError("write your pallas kernel here")


# speedup vs baseline: 17.3535x; 17.3535x over previous
import functools

import jax
import jax.numpy as jnp
from jax import lax
from jax.experimental import pallas as pl
from jax.experimental.pallas import tpu as pltpu
from jax.experimental.pallas import tpu_sc as plsc

_N = 10000
_E = 320000
_F = 128
_NC = 2
_NS = 16
_NW = _NC * _NS
_EPW = _E // _NW
_B = 80
_CH = _EPW // _B
_RPT = _N // _NS

_mesh = plsc.VectorSubcoreMesh(core_axis_name="c", subcore_axis_name="s")


def _sc_edge_aggregate(y, src_r, dst_r, zrows):
    @functools.partial(
        pl.kernel,
        out_type=jax.ShapeDtypeStruct((_NC, _NS, _RPT, _F), jnp.float32),
        mesh=_mesh,
        scratch_types=[
            pltpu.VMEM_SHARED((_N, _F), jnp.float32),
            pltpu.VMEM((_CH, _B), jnp.int32),
            pltpu.VMEM((_CH, _B), jnp.int32),
            pltpu.VMEM((_B, _F), jnp.float32),
            pltpu.SemaphoreType.DMA,
        ],
    )
    def k(y_hbm, src_hbm, dst_hbm, z_hbm, out_hbm, acc, src_v, dst_v, buf, sem):
        c = lax.axis_index("c")
        s = lax.axis_index("s")
        wid = s * _NC + c
        pltpu.sync_copy(z_hbm, acc.at[pl.ds(s * _RPT, _RPT)])
        pltpu.sync_copy(src_hbm.at[wid], src_v)
        pltpu.sync_copy(dst_hbm.at[wid], dst_v)
        plsc.subcore_barrier()

        def body(j, carry):
            pltpu.async_copy(y_hbm.at[src_v.at[j]], buf, sem).wait()
            pltpu.sync_copy(buf, acc.at[dst_v.at[j]], add=True)
            return carry

        lax.fori_loop(0, _CH, body, 0)
        plsc.subcore_barrier()
        pltpu.sync_copy(acc.at[pl.ds(s * _RPT, _RPT)], out_hbm.at[c, s])

    return k(y, src_r, dst_r, zrows).reshape(_NC, _N, _F)


def _sc_degree_hist(dst_r, e0rows, zrows):
    @functools.partial(
        pl.kernel,
        out_type=jax.ShapeDtypeStruct((_NC, _NS, _RPT, 16), jnp.float32),
        mesh=_mesh,
        scratch_types=[
            pltpu.VMEM_SHARED((_N, 16), jnp.float32),
            pltpu.VMEM((_CH, _B), jnp.int32),
            pltpu.VMEM((_B, 16), jnp.float32),
        ],
    )
    def k(dst_hbm, e0_hbm, z_hbm, out_hbm, acc, dst_v, buf):
        c = lax.axis_index("c")
        s = lax.axis_index("s")
        wid = s * _NC + c
        pltpu.sync_copy(z_hbm, acc.at[pl.ds(s * _RPT, _RPT)])
        pltpu.sync_copy(e0_hbm, buf)
        pltpu.sync_copy(dst_hbm.at[wid], dst_v)
        plsc.subcore_barrier()

        def body(j, carry):
            pltpu.sync_copy(buf, acc.at[dst_v.at[j]], add=True)
            return carry

        lax.fori_loop(0, _CH, body, 0)
        plsc.subcore_barrier()
        pltpu.sync_copy(acc.at[pl.ds(s * _RPT, _RPT)], out_hbm.at[c, s])

    return k(dst_r, e0rows, zrows).reshape(_NC, _N, 16)


def _dinv_block(ha, hb):
    deg = (jnp.sum(ha, axis=1, keepdims=True)
           + jnp.sum(hb, axis=1, keepdims=True) + 1.0)
    return lax.rsqrt(deg)


_GRID = 10
_BR = _N // _GRID


def _tc_first(x, W1, ha, hb):
    def body(x_ref, w_ref, ha_ref, hb_ref, y_ref):
        dinv = _dinv_block(ha_ref[...], hb_ref[...])
        y_ref[...] = jnp.dot(x_ref[...], w_ref[...],
                             preferred_element_type=jnp.float32) * dinv

    return pl.pallas_call(
        body,
        grid=(_GRID,),
        in_specs=[
            pl.BlockSpec((_BR, _F), lambda i: (i, 0)),
            pl.BlockSpec((_F, _F), lambda i: (0, 0)),
            pl.BlockSpec((_BR, 16), lambda i: (i, 0)),
            pl.BlockSpec((_BR, 16), lambda i: (i, 0)),
        ],
        out_specs=pl.BlockSpec((_BR, _F), lambda i: (i, 0)),
        out_shape=jax.ShapeDtypeStruct((_N, _F), jnp.float32),
    )(x, W1, ha, hb)


def _tc_mid(agga, aggb, y1, ha, hb, W2, b1):
    def body(aa_ref, ab_ref, y1_ref, ha_ref, hb_ref, w_ref, b_ref, y2_ref):
        dinv = _dinv_block(ha_ref[...], hb_ref[...])
        h1 = jax.nn.sigmoid(
            (aa_ref[...] + ab_ref[...] + y1_ref[...]) * dinv + b_ref[...])
        y2_ref[...] = jnp.dot(h1, w_ref[...],
                              preferred_element_type=jnp.float32) * dinv

    return pl.pallas_call(
        body,
        grid=(_GRID,),
        in_specs=[
            pl.BlockSpec((_BR, _F), lambda i: (i, 0)),
            pl.BlockSpec((_BR, _F), lambda i: (i, 0)),
            pl.BlockSpec((_BR, _F), lambda i: (i, 0)),
            pl.BlockSpec((_BR, 16), lambda i: (i, 0)),
            pl.BlockSpec((_BR, 16), lambda i: (i, 0)),
            pl.BlockSpec((_F, _F), lambda i: (0, 0)),
            pl.BlockSpec((1, _F), lambda i: (0, 0)),
        ],
        out_specs=pl.BlockSpec((_BR, _F), lambda i: (i, 0)),
        out_shape=jax.ShapeDtypeStruct((_N, _F), jnp.float32),
    )(agga, aggb, y1, ha, hb, W2, b1)


def _tc_last(agga, aggb, y2, ha, hb, b2, batch_r,
             Wil, bil, Whl1, bhl1, Wol, bol):
    def body(aa_ref, ab_ref, y2_ref, ha_ref, hb_ref, b2_ref, batch_ref,
             wil_ref, bil_ref, whl_ref, bhl_ref, wol_ref, bol_ref, out_ref):
        dinv = _dinv_block(ha_ref[...], hb_ref[...])
        h2 = jax.nn.relu(
            (aa_ref[...] + ab_ref[...] + y2_ref[...]) * dinv + b2_ref[...])
        gid = lax.broadcasted_iota(jnp.int32, (64, _N), 0)
        m = (batch_ref[...] == gid).astype(jnp.float32)
        sums = jnp.dot(m, h2, preferred_element_type=jnp.float32)
        cnts = jnp.sum(m, axis=1, keepdims=True)
        pooled = sums / jnp.maximum(cnts, 1.0)
        o = jax.nn.sigmoid(jnp.dot(pooled, wil_ref[...],
                                   preferred_element_type=jnp.float32)
                           + bil_ref[...])
        o = jax.nn.relu(jnp.dot(o, whl_ref[...],
                                preferred_element_type=jnp.float32)
                        + bhl_ref[...])
        out_ref[...] = (jnp.dot(o, wol_ref[...],
                                preferred_element_type=jnp.float32)
                        + bol_ref[...])

    return pl.pallas_call(
        body,
        out_shape=jax.ShapeDtypeStruct((64, 1), jnp.float32),
    )(agga, aggb, y2, ha, hb, b2, batch_r, Wil, bil, Whl1, bhl1, Wol, bol)


def kernel(x, edge_index, batch, W1, b1, W2, b2, Wil, bil, Whl1, bhl1, Wol, bol):
    src_r = edge_index[0].reshape(_NW, _CH, _B)
    dst_r = edge_index[1].reshape(_NW, _CH, _B)
    z128 = jnp.zeros((_RPT, _F), jnp.float32)
    z16 = jnp.zeros((_RPT, 16), jnp.float32)
    e0 = jnp.zeros((_B, 16), jnp.float32).at[:, 0].set(1.0)

    hist = _sc_degree_hist(dst_r, e0, z16)
    ha = hist[0]
    hb = hist[1]
    y1 = _tc_first(x, W1, ha, hb)
    agg1 = _sc_edge_aggregate(y1, src_r, dst_r, z128)
    y2 = _tc_mid(agg1[0], agg1[1], y1, ha, hb, W2, b1.reshape(1, _F))
    agg2 = _sc_edge_aggregate(y2, src_r, dst_r, z128)
    return _tc_last(agg2[0], agg2[1], y2, ha, hb, b2.reshape(1, _F),
                    batch.reshape(1, _N).astype(jnp.int32),
                    Wil, bil.reshape(1, 64), Whl1, bhl1.reshape(1, 16),
                    Wol, bol.reshape(1, 1))


# speedup vs baseline: 20.6803x; 1.1917x over previous
import functools

import jax
import jax.numpy as jnp
from jax import lax
from jax.experimental import pallas as pl
from jax.experimental.pallas import tpu as pltpu
from jax.experimental.pallas import tpu_sc as plsc

_N = 10000
_E = 320000
_F = 128
_NC = 2
_NS = 16
_NW = _NC * _NS
_EPW = _E // _NW
_B = 80
_CH = _EPW // _B
_G = 25
_NG = _CH // _G
_RPT = _N // _NS

_mesh = plsc.VectorSubcoreMesh(core_axis_name="c", subcore_axis_name="s")


def _sc_edge_aggregate(y, src_r, dst_r, zrows):
    @functools.partial(
        pl.kernel,
        out_type=jax.ShapeDtypeStruct((_NC, _NS, _RPT, _F), jnp.float32),
        mesh=_mesh,
        scratch_types=[
            pltpu.VMEM_SHARED((_N, _F), jnp.float32),
            pltpu.VMEM((_G, _B), jnp.int32),
            pltpu.VMEM((_G, _B), jnp.int32),
            pltpu.VMEM((_B, _F), jnp.float32),
            pltpu.VMEM((_B, _F), jnp.float32),
            pltpu.SemaphoreType.DMA,
            pltpu.SemaphoreType.DMA,
        ],
    )
    def k(y_hbm, src_hbm, dst_hbm, z_hbm, out_hbm, acc, src_v, dst_v,
          buf0, buf1, sem0, sem1):
        c = lax.axis_index("c")
        s = lax.axis_index("s")
        wid = s * _NC + c
        pltpu.sync_copy(z_hbm, acc.at[pl.ds(s * _RPT, _RPT)])
        plsc.subcore_barrier()

        def gather(j, buf, sem):
            return pltpu.make_async_copy(y_hbm.at[src_v.at[j]], buf, sem)

        def group(g, carry):
            pltpu.sync_copy(src_hbm.at[wid, g], src_v)
            pltpu.sync_copy(dst_hbm.at[wid, g], dst_v)
            gather(0, buf0, sem0).start()

            def body(k2, c2):
                ja = 2 * k2
                jb = ja + 1
                gather(ja, buf0, sem0).wait()
                gather(jb, buf1, sem1).start()
                pltpu.sync_copy(buf0, acc.at[dst_v.at[ja]], add=True)
                gather(jb, buf1, sem1).wait()
                gather(jb + 1, buf0, sem0).start()
                pltpu.sync_copy(buf1, acc.at[dst_v.at[jb]], add=True)
                return c2

            lax.fori_loop(0, (_G - 1) // 2, body, 0)
            gather(_G - 1, buf0, sem0).wait()
            pltpu.sync_copy(buf0, acc.at[dst_v.at[_G - 1]], add=True)
            return carry

        lax.fori_loop(0, _NG, group, 0)
        plsc.subcore_barrier()
        pltpu.sync_copy(acc.at[pl.ds(s * _RPT, _RPT)], out_hbm.at[c, s])

    return k(y, src_r, dst_r, zrows).reshape(_NC, _N, _F)


def _sc_degree_hist(dst_r, e0rows, zrows):
    @functools.partial(
        pl.kernel,
        out_type=jax.ShapeDtypeStruct((_NC, _NS, _RPT, 16), jnp.float32),
        mesh=_mesh,
        scratch_types=[
            pltpu.VMEM_SHARED((_N, 16), jnp.float32),
            pltpu.VMEM((_NG, _G, _B), jnp.int32),
            pltpu.VMEM((_B, 16), jnp.float32),
        ],
    )
    def k(dst_hbm, e0_hbm, z_hbm, out_hbm, acc, dst_v, buf):
        c = lax.axis_index("c")
        s = lax.axis_index("s")
        wid = s * _NC + c
        pltpu.sync_copy(z_hbm, acc.at[pl.ds(s * _RPT, _RPT)])
        pltpu.sync_copy(e0_hbm, buf)
        pltpu.sync_copy(dst_hbm.at[wid], dst_v)
        plsc.subcore_barrier()

        def body(j, carry):
            pltpu.sync_copy(buf, acc.at[dst_v.at[j // _G, j % _G]], add=True)
            return carry

        lax.fori_loop(0, _CH, body, 0)
        plsc.subcore_barrier()
        pltpu.sync_copy(acc.at[pl.ds(s * _RPT, _RPT)], out_hbm.at[c, s])

    return k(dst_r, e0rows, zrows).reshape(_NC, _N, 16)


def _dinv_block(ha, hb):
    deg = (jnp.sum(ha, axis=1, keepdims=True)
           + jnp.sum(hb, axis=1, keepdims=True) + 1.0)
    return lax.rsqrt(deg)


_GRID = 10
_BR = _N // _GRID


def _tc_first(x, W1, ha, hb):
    def body(x_ref, w_ref, ha_ref, hb_ref, y_ref):
        dinv = _dinv_block(ha_ref[...], hb_ref[...])
        y_ref[...] = jnp.dot(x_ref[...], w_ref[...],
                             preferred_element_type=jnp.float32) * dinv

    return pl.pallas_call(
        body,
        grid=(_GRID,),
        in_specs=[
            pl.BlockSpec((_BR, _F), lambda i: (i, 0)),
            pl.BlockSpec((_F, _F), lambda i: (0, 0)),
            pl.BlockSpec((_BR, 16), lambda i: (i, 0)),
            pl.BlockSpec((_BR, 16), lambda i: (i, 0)),
        ],
        out_specs=pl.BlockSpec((_BR, _F), lambda i: (i, 0)),
        out_shape=jax.ShapeDtypeStruct((_N, _F), jnp.float32),
    )(x, W1, ha, hb)


def _tc_mid(agga, aggb, y1, ha, hb, W2, b1):
    def body(aa_ref, ab_ref, y1_ref, ha_ref, hb_ref, w_ref, b_ref, y2_ref):
        dinv = _dinv_block(ha_ref[...], hb_ref[...])
        h1 = jax.nn.sigmoid(
            (aa_ref[...] + ab_ref[...] + y1_ref[...]) * dinv + b_ref[...])
        y2_ref[...] = jnp.dot(h1, w_ref[...],
                              preferred_element_type=jnp.float32) * dinv

    return pl.pallas_call(
        body,
        grid=(_GRID,),
        in_specs=[
            pl.BlockSpec((_BR, _F), lambda i: (i, 0)),
            pl.BlockSpec((_BR, _F), lambda i: (i, 0)),
            pl.BlockSpec((_BR, _F), lambda i: (i, 0)),
            pl.BlockSpec((_BR, 16), lambda i: (i, 0)),
            pl.BlockSpec((_BR, 16), lambda i: (i, 0)),
            pl.BlockSpec((_F, _F), lambda i: (0, 0)),
            pl.BlockSpec((1, _F), lambda i: (0, 0)),
        ],
        out_specs=pl.BlockSpec((_BR, _F), lambda i: (i, 0)),
        out_shape=jax.ShapeDtypeStruct((_N, _F), jnp.float32),
    )(agga, aggb, y1, ha, hb, W2, b1)


def _tc_last(agga, aggb, y2, ha, hb, b2, batch_r,
             Wil, bil, Whl1, bhl1, Wol, bol):
    def body(aa_ref, ab_ref, y2_ref, ha_ref, hb_ref, b2_ref, batch_ref,
             wil_ref, bil_ref, whl_ref, bhl_ref, wol_ref, bol_ref, out_ref):
        dinv = _dinv_block(ha_ref[...], hb_ref[...])
        h2 = jax.nn.relu(
            (aa_ref[...] + ab_ref[...] + y2_ref[...]) * dinv + b2_ref[...])
        gid = lax.broadcasted_iota(jnp.int32, (64, _N), 0)
        m = (batch_ref[...] == gid).astype(jnp.float32)
        sums = jnp.dot(m, h2, preferred_element_type=jnp.float32)
        cnts = jnp.sum(m, axis=1, keepdims=True)
        pooled = sums / jnp.maximum(cnts, 1.0)
        o = jax.nn.sigmoid(jnp.dot(pooled, wil_ref[...],
                                   preferred_element_type=jnp.float32)
                           + bil_ref[...])
        o = jax.nn.relu(jnp.dot(o, whl_ref[...],
                                preferred_element_type=jnp.float32)
                        + bhl_ref[...])
        out_ref[...] = (jnp.dot(o, wol_ref[...],
                                preferred_element_type=jnp.float32)
                        + bol_ref[...])

    return pl.pallas_call(
        body,
        out_shape=jax.ShapeDtypeStruct((64, 1), jnp.float32),
    )(agga, aggb, y2, ha, hb, b2, batch_r, Wil, bil, Whl1, bhl1, Wol, bol)


def kernel(x, edge_index, batch, W1, b1, W2, b2, Wil, bil, Whl1, bhl1, Wol, bol):
    src_r = edge_index[0].reshape(_NW, _NG, _G, _B)
    dst_r = edge_index[1].reshape(_NW, _NG, _G, _B)
    z128 = jnp.zeros((_RPT, _F), jnp.float32)
    z16 = jnp.zeros((_RPT, 16), jnp.float32)
    e0 = jnp.zeros((_B, 16), jnp.float32).at[:, 0].set(1.0)

    hist = _sc_degree_hist(dst_r, e0, z16)
    ha = hist[0]
    hb = hist[1]
    y1 = _tc_first(x, W1, ha, hb)
    agg1 = _sc_edge_aggregate(y1, src_r, dst_r, z128)
    y2 = _tc_mid(agg1[0], agg1[1], y1, ha, hb, W2, b1.reshape(1, _F))
    agg2 = _sc_edge_aggregate(y2, src_r, dst_r, z128)
    return _tc_last(agg2[0], agg2[1], y2, ha, hb, b2.reshape(1, _F),
                    batch.reshape(1, _N).astype(jnp.int32),
                    Wil, bil.reshape(1, 64), Whl1, bhl1.reshape(1, 16),
                    Wol, bol.reshape(1, 1))
